# Initial kernel scaffold; baseline (speedup 1.0000x reference)
import jax
import jax.numpy as jnp
from jax.experimental import pallas as pl


def kernel(x, edge_index, edge_attr, params):
    raise NotImplementedError("write your pallas kernel here")


# speedup vs baseline: 2.0243x; 2.0243x over previous
import functools

import jax
import jax.numpy as jnp
from jax import lax
from jax.experimental import pallas as pl
from jax.experimental.pallas import tpu as pltpu
from jax.experimental.pallas import tpu_sc as plsc

F32 = jnp.float32

NUM_SC = 2
NUM_SUBCORES = 16
NW = NUM_SC * NUM_SUBCORES
CH = 80


def _gelu(x):
    return jax.nn.gelu(x, approximate=True)


def _mlp2_body(x_ref, w1_ref, b1_ref, w2_ref, b2_ref, o_ref):
    h = jnp.dot(x_ref[...], w1_ref[...], preferred_element_type=F32)
    h = _gelu(h + b1_ref[...])
    o_ref[...] = jnp.dot(h, w2_ref[...], preferred_element_type=F32) + b2_ref[...]


def _mlp2_tc(x, w1, b1, w2, b2, bn):
    R, K = x.shape
    H1 = w1.shape[1]
    H2 = w2.shape[1]
    return pl.pallas_call(
        _mlp2_body,
        grid=(R // bn,),
        in_specs=[
            pl.BlockSpec((bn, K), lambda i: (i, 0)),
            pl.BlockSpec((K, H1), lambda i: (0, 0)),
            pl.BlockSpec((1, H1), lambda i: (0, 0)),
            pl.BlockSpec((H1, H2), lambda i: (0, 0)),
            pl.BlockSpec((1, H2), lambda i: (0, 0)),
        ],
        out_specs=pl.BlockSpec((bn, H2), lambda i: (i, 0)),
        out_shape=jax.ShapeDtypeStruct((R, H2), F32),
    )(x, w1, b1.reshape(1, -1), w2, b2.reshape(1, -1))


def _edge_body(ea_ref, hs_ref, w1_ref, b1_ref, w2_ref, b2_ref, o_ref):
    t = jnp.dot(ea_ref[...], w1_ref[...], preferred_element_type=F32)
    t = _gelu(t + b1_ref[...])
    ew = jnp.dot(t, w2_ref[...], preferred_element_type=F32) + b2_ref[...]
    o_ref[...] = ew * hs_ref[...]


def _edge_mul_tc(ea, hsrc, w1, b1, w2, b2, bn):
    E, K = ea.shape
    H1 = w1.shape[1]
    H = w2.shape[1]
    return pl.pallas_call(
        _edge_body,
        grid=(E // bn,),
        in_specs=[
            pl.BlockSpec((bn, K), lambda i: (i, 0)),
            pl.BlockSpec((bn, H), lambda i: (i, 0)),
            pl.BlockSpec((K, H1), lambda i: (0, 0)),
            pl.BlockSpec((1, H1), lambda i: (0, 0)),
            pl.BlockSpec((H1, H), lambda i: (0, 0)),
            pl.BlockSpec((1, H), lambda i: (0, 0)),
        ],
        out_specs=pl.BlockSpec((bn, H), lambda i: (i, 0)),
        out_shape=jax.ShapeDtypeStruct((E, H), F32),
    )(ea, hsrc, w1, b1.reshape(1, -1), w2, b2.reshape(1, -1))


def _update_body(h_ref, a0_ref, a1_ref, c0_ref, c1_ref, gwh_ref, gwa_ref,
                 gb_ref, g_ref, b_ref, o_ref):
    h = h_ref[...]
    cnt = c0_ref[:, 0:1] + c1_ref[:, 0:1]
    cnt = jnp.maximum(cnt, 1.0)
    agg = (a0_ref[...] + a1_ref[...]) / cnt
    z = (jnp.dot(h, gwh_ref[...], preferred_element_type=F32)
         + jnp.dot(agg, gwa_ref[...], preferred_element_type=F32)
         + gb_ref[...])
    gate = jax.nn.sigmoid(z)
    hn = h + gate * agg
    mu = jnp.mean(hn, axis=-1, keepdims=True)
    var = jnp.mean((hn - mu) ** 2, axis=-1, keepdims=True)
    o_ref[...] = (hn - mu) / jnp.sqrt(var + 1e-5) * g_ref[...] + b_ref[...]


def _node_update_tc(h, agg0, agg1, c0, c1, gwh, gwa, gb, ln_g, ln_b, bn):
    R, H = h.shape
    return pl.pallas_call(
        _update_body,
        grid=(R // bn,),
        in_specs=[
            pl.BlockSpec((bn, H), lambda i: (i, 0)),
            pl.BlockSpec((bn, H), lambda i: (i, 0)),
            pl.BlockSpec((bn, H), lambda i: (i, 0)),
            pl.BlockSpec((bn, 16), lambda i: (i, 0)),
            pl.BlockSpec((bn, 16), lambda i: (i, 0)),
            pl.BlockSpec((H, H), lambda i: (0, 0)),
            pl.BlockSpec((H, H), lambda i: (0, 0)),
            pl.BlockSpec((1, H), lambda i: (0, 0)),
            pl.BlockSpec((1, H), lambda i: (0, 0)),
            pl.BlockSpec((1, H), lambda i: (0, 0)),
        ],
        out_specs=pl.BlockSpec((bn, H), lambda i: (i, 0)),
        out_shape=jax.ShapeDtypeStruct((R, H), F32),
    )(h, agg0, agg1, c0, c1, gwh, gwa, gb.reshape(1, -1),
      ln_g.reshape(1, -1), ln_b.reshape(1, -1))


def _sc_gather(m, src):
    E = src.shape[0]
    H = m.shape[1]
    per = E // NW
    nch = per // CH
    mesh = plsc.VectorSubcoreMesh(core_axis_name="c", subcore_axis_name="s")

    @functools.partial(
        pl.kernel, mesh=mesh,
        out_type=jax.ShapeDtypeStruct((E, H), F32),
        scratch_types=[
            pltpu.VMEM((CH,), jnp.int32),
            pltpu.VMEM((CH, H), F32),
        ],
    )
    def k(m_hbm, src_hbm, out_hbm, idx_v, rows_v):
        wid = lax.axis_index("s") * NUM_SC + lax.axis_index("c")
        base = wid * per

        @pl.loop(0, nch)
        def _(i):
            e0 = base + i * CH
            pltpu.sync_copy(src_hbm.at[pl.ds(e0, CH)], idx_v)
            pltpu.sync_copy(m_hbm.at[idx_v], rows_v)
            pltpu.sync_copy(rows_v, out_hbm.at[pl.ds(e0, CH)])

    return k(m, src)


def _sc_scatter_add(msg, dst, np_rows):
    E, H = msg.shape
    per = E // NW
    nch = per // CH
    rp = np_rows // NUM_SUBCORES
    nz = rp // CH
    mesh = plsc.VectorSubcoreMesh(core_axis_name="c", subcore_axis_name="s")

    @functools.partial(
        pl.kernel, mesh=mesh,
        out_type=jax.ShapeDtypeStruct((NUM_SC, np_rows, H), F32),
        scratch_types=[
            pltpu.VMEM((CH,), jnp.int32),
            pltpu.VMEM((CH, H), F32),
            pltpu.VMEM_SHARED((np_rows, H), F32),
        ],
    )
    def k(msg_hbm, dst_hbm, out_hbm, idx_v, rows_v, acc_sh):
        cid = lax.axis_index("c")
        sid = lax.axis_index("s")
        wid = sid * NUM_SC + cid

        @pl.loop(0, CH)
        def _(r):
            @pl.loop(0, H, step=16)
            def _(q):
                rows_v[r, pl.ds(q, 16)] = jnp.zeros((16,), F32)

        @pl.loop(0, nz)
        def _(j):
            pltpu.sync_copy(rows_v, acc_sh.at[pl.ds(sid * rp + j * CH, CH)])

        plsc.subcore_barrier()

        @pl.loop(0, nch)
        def _(i):
            e0 = wid * per + i * CH
            pltpu.sync_copy(dst_hbm.at[pl.ds(e0, CH)], idx_v)
            pltpu.sync_copy(msg_hbm.at[pl.ds(e0, CH)], rows_v)
            pltpu.sync_copy(rows_v, acc_sh.at[idx_v], add=True)

        plsc.subcore_barrier()
        pltpu.sync_copy(acc_sh.at[pl.ds(sid * rp, rp)],
                        out_hbm.at[cid, pl.ds(sid * rp, rp)])

    return k(msg, dst)


def _sc_count(dst, np_rows):
    E = dst.shape[0]
    W = 16
    per = E // NW
    nch = per // CH
    rp = np_rows // NUM_SUBCORES
    nz = rp // CH
    mesh = plsc.VectorSubcoreMesh(core_axis_name="c", subcore_axis_name="s")

    @functools.partial(
        pl.kernel, mesh=mesh,
        out_type=jax.ShapeDtypeStruct((NUM_SC, np_rows, W), F32),
        scratch_types=[
            pltpu.VMEM((CH,), jnp.int32),
            pltpu.VMEM((CH, W), F32),
            pltpu.VMEM_SHARED((np_rows, W), F32),
        ],
    )
    def k(dst_hbm, out_hbm, idx_v, buf_v, acc_sh):
        cid = lax.axis_index("c")
        sid = lax.axis_index("s")
        wid = sid * NUM_SC + cid

        @pl.loop(0, CH)
        def _(r):
            buf_v[r, pl.ds(0, W)] = jnp.zeros((W,), F32)

        @pl.loop(0, nz)
        def _(j):
            pltpu.sync_copy(buf_v, acc_sh.at[pl.ds(sid * rp + j * CH, CH)])

        plsc.subcore_barrier()

        @pl.loop(0, CH)
        def _(r):
            buf_v[r, pl.ds(0, W)] = jnp.ones((W,), F32)

        @pl.loop(0, nch)
        def _(i):
            e0 = wid * per + i * CH
            pltpu.sync_copy(dst_hbm.at[pl.ds(e0, CH)], idx_v)
            pltpu.sync_copy(buf_v, acc_sh.at[idx_v], add=True)

        plsc.subcore_barrier()
        pltpu.sync_copy(acc_sh.at[pl.ds(sid * rp, rp)],
                        out_hbm.at[cid, pl.ds(sid * rp, rp)])

    return k(dst)


def kernel(x, edge_index, edge_attr, params):
    N, AF = x.shape
    E = edge_index.shape[1]
    H = 128

    np_rows = ((N + NUM_SUBCORES * CH - 1) // (NUM_SUBCORES * CH)) * (NUM_SUBCORES * CH)
    bn_nodes = np_rows // 10
    bn_edges = 2000

    src = edge_index[0].astype(jnp.int32)
    dst = edge_index[1].astype(jnp.int32)

    w1, b1, w2, b2 = params['enc']
    kpad = 32 - AF
    xp = jnp.pad(x, ((0, np_rows - N), (0, kpad)))
    w1p = jnp.pad(w1, ((0, kpad), (0, 0)))
    h = _mlp2_tc(xp, w1p, b1, w2, b2, bn_nodes)

    cp = _sc_count(dst, np_rows)
    c0 = cp[0]
    c1 = cp[1]

    eapad = 16 - edge_attr.shape[1]
    eap = jnp.pad(edge_attr, ((0, 0), (0, eapad)))

    for lp in params['layers']:
        mW1, mb1, mW2, mb2 = lp['msg']
        eW1, eb1, eW2, eb2 = lp['edge']
        m = _mlp2_tc(h, mW1, mb1, mW2, mb2, bn_nodes)
        hsrc = _sc_gather(m, src)
        eW1p = jnp.pad(eW1, ((0, eapad), (0, 0)))
        msg = _edge_mul_tc(eap, hsrc, eW1p, eb1, eW2, eb2, bn_edges)
        aggp = _sc_scatter_add(msg, dst, np_rows)
        gW = lp['gate_W']
        h = _node_update_tc(h, aggp[0], aggp[1], c0, c1,
                            gW[:H], gW[H:], lp['gate_b'],
                            lp['ln_g'], lp['ln_b'], bn_nodes)

    outs = []
    for name in ('ion', 'pka', 'acid'):
        hW1, hb1, hW2, hb2 = params[name]
        hW2p = jnp.pad(hW2, ((0, 0), (0, H - 1)))
        hb2p = jnp.pad(hb2, ((0, H - 1)))
        o = _mlp2_tc(h, hW1, hb1, hW2p, hb2p, bn_nodes)
        outs.append(o[:N, 0:1])

    return tuple(outs)
